# baseline (speedup 1.0000x reference)
import functools

import jax
import jax.numpy as jnp
from jax import lax
from jax.experimental import pallas as pl
from jax.experimental.pallas import tpu as pltpu
from jax.experimental.pallas import tpu_sc as plsc

VOCAB = 32000
EMB = 384
HID = 512
G4 = 4 * HID
LAYERS = 3
B = 16
T = 128
BT = B * T

_NC, _NS = 2, 16
_NW = _NC * _NS
_BPW = BT // _NW


def _gather_body(table_hbm, idx_hbm, out_hbm, idx_v, rows_v, sem):
    wid = lax.axis_index("s") * _NC + lax.axis_index("c")
    base = wid * _BPW
    pltpu.sync_copy(idx_hbm.at[pl.ds(base, _BPW)], idx_v)
    pltpu.async_copy(table_hbm.at[idx_v], rows_v, sem).wait()
    pltpu.sync_copy(rows_v, out_hbm.at[pl.ds(base, _BPW)])


def _sc_gather(table, idx):
    mesh = plsc.VectorSubcoreMesh(core_axis_name="c", subcore_axis_name="s")
    k = functools.partial(
        pl.kernel,
        mesh=mesh,
        out_type=jax.ShapeDtypeStruct((BT, EMB), jnp.float32),
        scratch_types=[
            pltpu.VMEM((_BPW,), jnp.int32),
            pltpu.VMEM((_BPW, EMB), jnp.float32),
            pltpu.SemaphoreType.DMA,
        ],
    )(_gather_body)
    return k(table, idx)


_DN = (((1,), (1,)), ((), ()))
_DNT = (((1,), (0,)), ((), ()))


def _lstm_body(xs_ref, h0_ref, c0_ref,
               wi0, wh0, bb0, wi1, wh1, bb1, wi2, wh2, bb2,
               lng, lnb, out_ref, ht_ref, ct_ref, seq_ref, proj_ref):
    bf = jnp.bfloat16

    def gates(g, c):
        z = jnp.tanh(g)
        i = 0.5 + 0.5 * z[:, 0:HID]
        f = 0.5 + 0.5 * z[:, HID:2 * HID]
        gg = z[:, 2 * HID:3 * HID]
        o = 0.5 + 0.5 * z[:, 3 * HID:4 * HID]
        c2 = f * c + i * gg
        return o * jnp.tanh(c2), c2

    for l, (wi, wh, bb) in enumerate(((wi0, wh0, bb0),
                                      (wi1, wh1, bb1),
                                      (wi2, wh2, bb2))):
        src = xs_ref if l == 0 else seq_ref
        proj_ref[...] = lax.dot_general(
            src[...].astype(bf), wi[...], _DN,
            preferred_element_type=jnp.float32) + bb[...]
        wh_v = wh[...]

        def step(t, carry, wh_v=wh_v):
            h, c = carry
            g = proj_ref[pl.ds(t * B, B), :] + lax.dot_general(
                h.astype(bf), wh_v, _DNT,
                preferred_element_type=jnp.float32)
            h2, c2 = gates(g, c)
            seq_ref[pl.ds(t * B, B), :] = h2
            return h2, c2

        h_n, c_n = lax.fori_loop(0, T, step, (h0_ref[l], c0_ref[l]))
        ht_ref[l] = h_n
        ct_ref[l] = c_n

    y = seq_ref[...]
    mu = jnp.mean(y, axis=1, keepdims=True)
    var = jnp.mean((y - mu) ** 2, axis=1, keepdims=True)
    out_ref[...] = ((y - mu) * lax.rsqrt(var + 1e-5) * lng[...]
                    + lnb[...]).astype(bf)


def _lstm_call(xs, h0, c0, wi0, wh0, bb0, wi1, wh1, bb1, wi2, wh2, bb2,
               lng, lnb):
    return pl.pallas_call(
        _lstm_body,
        out_shape=(
            jax.ShapeDtypeStruct((BT, HID), jnp.bfloat16),
            jax.ShapeDtypeStruct((LAYERS, B, HID), jnp.float32),
            jax.ShapeDtypeStruct((LAYERS, B, HID), jnp.float32),
        ),
        scratch_shapes=[
            pltpu.VMEM((BT, HID), jnp.float32),
            pltpu.VMEM((BT, G4), jnp.float32),
        ],
    )(xs, h0, c0, wi0, wh0, bb0, wi1, wh1, bb1, wi2, wh2, bb2, lng, lnb)


_VB = 1280


def _fc_body(x_ref, w_ref, b_ref, o_ref):
    o_ref[...] = lax.dot_general(
        x_ref[...], w_ref[...].astype(jnp.bfloat16),
        _DN, preferred_element_type=jnp.float32) + b_ref[...]


def _fc_call(xb, fc_w, fc_b):
    return pl.pallas_call(
        _fc_body,
        grid=(VOCAB // _VB,),
        in_specs=[
            pl.BlockSpec((BT, HID), lambda i: (0, 0)),
            pl.BlockSpec((_VB, HID), lambda i: (i, 0)),
            pl.BlockSpec((1, _VB), lambda i: (0, i)),
        ],
        out_specs=pl.BlockSpec((BT, _VB), lambda i: (0, i)),
        out_shape=jax.ShapeDtypeStruct((BT, VOCAB), jnp.float32),
    )(xb, fc_w, fc_b)


def kernel(x, h0, c0, emb, W_ih_0, W_hh_0, b_ih_0, b_hh_0,
           W_ih_1, W_hh_1, b_ih_1, b_hh_1, W_ih_2, W_hh_2, b_ih_2, b_hh_2,
           ln_g, ln_b, fc_W, fc_b):
    idx = x.T.reshape(BT).astype(jnp.int32)
    gath = _sc_gather(emb, idx)

    gsc = jnp.concatenate(
        [jnp.full((2 * HID,), 0.5, jnp.float32),
         jnp.ones((HID,), jnp.float32),
         jnp.full((HID,), 0.5, jnp.float32)])
    bb0 = ((b_ih_0 + b_hh_0) * gsc).reshape(1, G4)
    bb1 = ((b_ih_1 + b_hh_1) * gsc).reshape(1, G4)
    bb2 = ((b_ih_2 + b_hh_2) * gsc).reshape(1, G4)
    bf = jnp.bfloat16
    out_norm, h_t, c_t = _lstm_call(
        gath, h0, c0,
        (W_ih_0 * gsc[:, None]).astype(bf), (W_hh_0.T * gsc).astype(bf), bb0,
        (W_ih_1 * gsc[:, None]).astype(bf), (W_hh_1.T * gsc).astype(bf), bb1,
        (W_ih_2 * gsc[:, None]).astype(bf), (W_hh_2.T * gsc).astype(bf), bb2,
        ln_g.reshape(1, HID), ln_b.reshape(1, HID))

    xb = out_norm.reshape(T, B, HID).transpose(1, 0, 2).reshape(BT, HID)
    logits = _fc_call(xb, fc_W, fc_b.reshape(1, VOCAB))
    return logits.reshape(B, T, VOCAB), h_t, c_t


# speedup vs baseline: 1.0661x; 1.0661x over previous
import functools

import jax
import jax.numpy as jnp
from jax import lax
from jax.experimental import pallas as pl
from jax.experimental.pallas import tpu as pltpu
from jax.experimental.pallas import tpu_sc as plsc

VOCAB = 32000
EMB = 384
HID = 512
G4 = 4 * HID
LAYERS = 3
B = 16
T = 128
BT = B * T

_NC, _NS = 2, 16
_NW = _NC * _NS
_BPW = BT // _NW


def _gather_body(table_hbm, idx_hbm, out_hbm, idx_v, rows_v, sem):
    wid = lax.axis_index("s") * _NC + lax.axis_index("c")
    base = wid * _BPW
    pltpu.sync_copy(idx_hbm.at[pl.ds(base, _BPW)], idx_v)
    pltpu.async_copy(table_hbm.at[idx_v], rows_v, sem).wait()
    pltpu.sync_copy(rows_v, out_hbm.at[pl.ds(base, _BPW)])


def _sc_gather(table, idx):
    mesh = plsc.VectorSubcoreMesh(core_axis_name="c", subcore_axis_name="s")
    k = functools.partial(
        pl.kernel,
        mesh=mesh,
        out_type=jax.ShapeDtypeStruct((BT, EMB), jnp.float32),
        scratch_types=[
            pltpu.VMEM((_BPW,), jnp.int32),
            pltpu.VMEM((_BPW, EMB), jnp.float32),
            pltpu.SemaphoreType.DMA,
        ],
    )(_gather_body)
    return k(table, idx)


_DN = (((1,), (1,)), ((), ()))
_DNT = (((1,), (0,)), ((), ()))


def _lstm_body(xs_ref, h0_ref, c0_ref,
               wi0, wh0, bb0, wi1, wh1, bb1, wi2, wh2, bb2,
               lng, lnb, out_ref, ht_ref, ct_ref, seq_ref, proj_ref):
    bf = jnp.bfloat16
    gscale = jnp.concatenate(
        [jnp.full((1, 2 * HID), 0.5, jnp.float32),
         jnp.full((1, HID), 1.0, jnp.float32),
         jnp.full((1, HID), 0.5, jnp.float32)], axis=1)

    def gates(g, c):
        z = jnp.tanh(g * gscale)
        i = 0.5 + 0.5 * z[:, 0:HID]
        f = 0.5 + 0.5 * z[:, HID:2 * HID]
        gg = z[:, 2 * HID:3 * HID]
        o = 0.5 + 0.5 * z[:, 3 * HID:4 * HID]
        c2 = f * c + i * gg
        return o * jnp.tanh(c2), c2

    for l, (wi, wh, bb) in enumerate(((wi0, wh0, bb0),
                                      (wi1, wh1, bb1),
                                      (wi2, wh2, bb2))):
        src = xs_ref if l == 0 else seq_ref
        proj_ref[...] = lax.dot_general(
            src[...].astype(bf), wi[...], _DN,
            preferred_element_type=jnp.float32) + bb[...]
        wh_v = wh[...]

        def step(t, carry, wh_v=wh_v):
            h, c = carry
            g = proj_ref[pl.ds(t * B, B), :] + lax.dot_general(
                h.astype(bf), wh_v, _DNT,
                preferred_element_type=jnp.float32)
            h2, c2 = gates(g, c)
            seq_ref[pl.ds(t * B, B), :] = h2
            return h2, c2

        h_n, c_n = lax.fori_loop(0, T, step, (h0_ref[l], c0_ref[l]))
        ht_ref[l] = h_n
        ct_ref[l] = c_n

    y = seq_ref[...]
    mu = jnp.mean(y, axis=1, keepdims=True)
    var = jnp.mean((y - mu) ** 2, axis=1, keepdims=True)
    out_ref[...] = (y - mu) * lax.rsqrt(var + 1e-5) * lng[...] + lnb[...]


def _lstm_call(xs, h0, c0, wi0, wh0, bb0, wi1, wh1, bb1, wi2, wh2, bb2,
               lng, lnb):
    return pl.pallas_call(
        _lstm_body,
        out_shape=(
            jax.ShapeDtypeStruct((BT, HID), jnp.float32),
            jax.ShapeDtypeStruct((LAYERS, B, HID), jnp.float32),
            jax.ShapeDtypeStruct((LAYERS, B, HID), jnp.float32),
        ),
        scratch_shapes=[
            pltpu.VMEM((BT, HID), jnp.float32),
            pltpu.VMEM((BT, G4), jnp.float32),
        ],
    )(xs, h0, c0, wi0, wh0, bb0, wi1, wh1, bb1, wi2, wh2, bb2, lng, lnb)


_VB = 1280


def _fc_body(x_ref, w_ref, b_ref, o_ref):
    o_ref[...] = lax.dot_general(
        x_ref[...].astype(jnp.bfloat16), w_ref[...].astype(jnp.bfloat16),
        _DN, preferred_element_type=jnp.float32) + b_ref[...]


def _fc_call(xb, fc_w, fc_b):
    return pl.pallas_call(
        _fc_body,
        grid=(VOCAB // _VB,),
        in_specs=[
            pl.BlockSpec((BT, HID), lambda i: (0, 0)),
            pl.BlockSpec((_VB, HID), lambda i: (i, 0)),
            pl.BlockSpec((1, _VB), lambda i: (0, i)),
        ],
        out_specs=pl.BlockSpec((BT, _VB), lambda i: (0, i)),
        out_shape=jax.ShapeDtypeStruct((BT, VOCAB), jnp.float32),
    )(xb, fc_w, fc_b)


def kernel(x, h0, c0, emb, W_ih_0, W_hh_0, b_ih_0, b_hh_0,
           W_ih_1, W_hh_1, b_ih_1, b_hh_1, W_ih_2, W_hh_2, b_ih_2, b_hh_2,
           ln_g, ln_b, fc_W, fc_b):
    idx = x.T.reshape(BT).astype(jnp.int32)
    gath = _sc_gather(emb, idx)

    bb0 = (b_ih_0 + b_hh_0).reshape(1, G4)
    bb1 = (b_ih_1 + b_hh_1).reshape(1, G4)
    bb2 = (b_ih_2 + b_hh_2).reshape(1, G4)
    bf = jnp.bfloat16
    out_norm, h_t, c_t = _lstm_call(
        gath, h0, c0,
        W_ih_0.astype(bf), W_hh_0.T.astype(bf), bb0,
        W_ih_1.astype(bf), W_hh_1.T.astype(bf), bb1,
        W_ih_2.astype(bf), W_hh_2.T.astype(bf), bb2,
        ln_g.reshape(1, HID), ln_b.reshape(1, HID))

    xb = out_norm.reshape(T, B, HID).transpose(1, 0, 2).reshape(BT, HID)
    logits = _fc_call(xb, fc_W, fc_b.reshape(1, VOCAB))
    return logits.reshape(B, T, VOCAB), h_t, c_t
